# Initial kernel scaffold; baseline (speedup 1.0000x reference)
import functools

import jax
import jax.numpy as jnp
from jax.experimental import pallas as pl
from jax.experimental.pallas import tpu as pltpu

E = 8
D_MODEL = 2048
D_FF = 768


def _moe_body(x_ref, wg_ref, wgate_ref, wup_ref, wdown_ref,
              out_ref, logits_ref, wfull_ref, *, n_tok):
    e = pl.program_id(0)
    x = x_ref[...]

    @pl.when(e == 0)
    def _router():
        logits = jax.lax.dot_general(
            x, wg_ref[...], (((1,), (1,)), ((), ())),
            preferred_element_type=jnp.float32)
        logits_ref[...] = logits
        idx = jax.lax.broadcasted_iota(jnp.int32, (n_tok, E), 1)
        m1 = jnp.max(logits, axis=1, keepdims=True)
        i1 = -jnp.max(jnp.where(logits == m1, -idx, -E - 1), axis=1,
                      keepdims=True)
        masked = jnp.where(idx == i1, -jnp.inf, logits)
        m2 = jnp.max(masked, axis=1, keepdims=True)
        i2 = -jnp.max(jnp.where(masked == m2, -idx, -E - 1), axis=1,
                      keepdims=True)
        w1 = 1.0 / (1.0 + jnp.exp(m2 - m1))
        w2 = 1.0 - w1
        wfull_ref[...] = jnp.where(idx == i1, w1,
                                   jnp.where(idx == i2, w2, 0.0))
        out_ref[...] = jnp.zeros_like(out_ref)

    wg = wgate_ref[0]
    wu = wup_ref[0]
    wd = wdown_ref[0]
    g = jax.lax.dot_general(x, wg, (((1,), (1,)), ((), ())),
                            preferred_element_type=jnp.float32)
    u = jax.lax.dot_general(x, wu, (((1,), (1,)), ((), ())),
                            preferred_element_type=jnp.float32)
    h = (g * jax.lax.logistic(g)) * u
    y = jax.lax.dot_general(h, wd, (((1,), (1,)), ((), ())),
                            preferred_element_type=jnp.float32)
    idx = jax.lax.broadcasted_iota(jnp.int32, (n_tok, E), 1)
    w_col = jnp.sum(jnp.where(idx == e, wfull_ref[...], 0.0), axis=1,
                    keepdims=True)
    out_ref[...] += w_col * y


def kernel(hidden_states, Wg, W_gate, W_up, W_down):
    B, S, D = hidden_states.shape
    x = hidden_states.reshape(-1, D)
    T = x.shape[0]

    out, logits = pl.pallas_call(
        functools.partial(_moe_body, n_tok=T),
        grid=(E,),
        in_specs=[
            pl.BlockSpec((T, D), lambda e: (0, 0)),
            pl.BlockSpec((E, D), lambda e: (0, 0)),
            pl.BlockSpec((1, D_FF, D), lambda e: (e, 0, 0)),
            pl.BlockSpec((1, D_FF, D), lambda e: (e, 0, 0)),
            pl.BlockSpec((1, D, D_FF), lambda e: (e, 0, 0)),
        ],
        out_specs=[
            pl.BlockSpec((T, D), lambda e: (0, 0)),
            pl.BlockSpec((T, E), lambda e: (0, 0)),
        ],
        out_shape=[
            jax.ShapeDtypeStruct((T, D), jnp.float32),
            jax.ShapeDtypeStruct((T, E), jnp.float32),
        ],
        scratch_shapes=[pltpu.VMEM((T, E), jnp.float32)],
    )(x, Wg, W_gate, W_up, W_down)

    return out.reshape(B, S, D), logits


# speedup vs baseline: 1.5529x; 1.5529x over previous
import functools

import jax
import jax.numpy as jnp
from jax.experimental import pallas as pl
from jax.experimental.pallas import tpu as pltpu

E = 8
D_MODEL = 2048
D_FF = 768
FF_B = 256
TB = 512


def _moe_body(x_ref, wg_ref, wgate_ref, wup_ref, wdown_ref,
              out_ref, logits_ref, wfull_ref, *, n_tok):
    e = pl.program_id(0)
    f = pl.program_id(1)
    t = pl.program_id(2)

    @pl.when((e == 0) & (f == 0) & (t == 0))
    def _router():
        x = x_ref[...]
        logits = jax.lax.dot_general(
            x, wg_ref[...], (((1,), (1,)), ((), ())),
            preferred_element_type=jnp.float32)
        logits_ref[...] = logits
        idx = jax.lax.broadcasted_iota(jnp.int32, (n_tok, E), 1)
        m1 = jnp.max(logits, axis=1, keepdims=True)
        i1 = -jnp.max(jnp.where(logits == m1, -idx, -E - 1), axis=1,
                      keepdims=True)
        masked = jnp.where(idx == i1, -jnp.inf, logits)
        m2 = jnp.max(masked, axis=1, keepdims=True)
        i2 = -jnp.max(jnp.where(masked == m2, -idx, -E - 1), axis=1,
                      keepdims=True)
        w1 = 1.0 / (1.0 + jnp.exp(m2 - m1))
        w2 = 1.0 - w1
        wfull_ref[...] = jnp.where(idx == i1, w1,
                                   jnp.where(idx == i2, w2, 0.0))

    xt = x_ref[pl.ds(t * TB, TB), :]
    wg = wgate_ref[0]
    wu = wup_ref[0]
    wd = wdown_ref[0]
    g = jax.lax.dot_general(xt, wg, (((1,), (1,)), ((), ())),
                            preferred_element_type=jnp.float32)
    u = jax.lax.dot_general(xt, wu, (((1,), (1,)), ((), ())),
                            preferred_element_type=jnp.float32)
    h = (g * jax.lax.logistic(g)) * u
    y = jax.lax.dot_general(h, wd, (((1,), (1,)), ((), ())),
                            preferred_element_type=jnp.float32)
    idx = jax.lax.broadcasted_iota(jnp.int32, (TB, E), 1)
    wfull_t = wfull_ref[pl.ds(t * TB, TB), :]
    w_col = jnp.sum(jnp.where(idx == e, wfull_t, 0.0), axis=1,
                    keepdims=True)

    @pl.when((e == 0) & (f == 0))
    def _init():
        out_ref[pl.ds(t * TB, TB), :] = w_col * y

    @pl.when((e > 0) | (f > 0))
    def _acc():
        out_ref[pl.ds(t * TB, TB), :] += w_col * y


def kernel(hidden_states, Wg, W_gate, W_up, W_down):
    B, S, D = hidden_states.shape
    x = hidden_states.reshape(-1, D)
    T = x.shape[0]
    NF = D_FF // FF_B
    NT = T // TB

    out, logits = pl.pallas_call(
        functools.partial(_moe_body, n_tok=T),
        grid=(E, NF, NT),
        in_specs=[
            pl.BlockSpec((T, D), lambda e, f, t: (0, 0)),
            pl.BlockSpec((E, D), lambda e, f, t: (0, 0)),
            pl.BlockSpec((1, FF_B, D), lambda e, f, t: (e, f, 0)),
            pl.BlockSpec((1, FF_B, D), lambda e, f, t: (e, f, 0)),
            pl.BlockSpec((1, D, FF_B), lambda e, f, t: (e, 0, f)),
        ],
        out_specs=[
            pl.BlockSpec((T, D), lambda e, f, t: (0, 0)),
            pl.BlockSpec((T, E), lambda e, f, t: (0, 0)),
        ],
        out_shape=[
            jax.ShapeDtypeStruct((T, D), jnp.float32),
            jax.ShapeDtypeStruct((T, E), jnp.float32),
        ],
        scratch_shapes=[pltpu.VMEM((T, E), jnp.float32)],
    )(x, Wg, W_gate, W_up, W_down)

    return out.reshape(B, S, D), logits
